# Initial kernel scaffold; baseline (speedup 1.0000x reference)
import jax
import jax.numpy as jnp
from jax.experimental import pallas as pl


def kernel(x0, edge_index0, samples0, x1, edge_index1, samples1, W_z, b_z, W_r, b_r, W_h, b_h, lz_W, lz_b, lr_W, lr_b, lh_W, lh_b, fc1_W, fc1_b, fc2_W, fc2_b):
    raise NotImplementedError("write your pallas kernel here")


# speedup vs baseline: 65.5685x; 65.5685x over previous
import functools

import jax
import jax.numpy as jnp
from jax import lax
from jax.experimental import pallas as pl
from jax.experimental.pallas import tpu as pltpu
from jax.experimental.pallas import tpu_sc as plsc

N = 100000
E = 1600000
S = 10000
H = 32
HID = 128

NS = 16
NC = 2
LANES = 128
TN = 6272
NP = NS * TN
ER = 12544
EP = ER * LANES
TPT = ER // NS
KC = 16
NCH = TPT // KC
SP = 10240
SPW = SP // 8


_sc_mesh = plsc.VectorSubcoreMesh(core_axis_name="c", subcore_axis_name="s")


@functools.partial(
    pl.kernel,
    out_type=(
        jax.ShapeDtypeStruct((NC, NP), jnp.float32),
        jax.ShapeDtypeStruct((NC, NP), jnp.float32),
        jax.ShapeDtypeStruct((NC, NP), jnp.float32),
        jax.ShapeDtypeStruct((NC, NP), jnp.float32),
    ),
    mesh=_sc_mesh,
    compiler_params=pltpu.CompilerParams(needs_layout_passes=False, use_tc_tiling_on_sc=False),
    scratch_types=[
        pltpu.VMEM_SHARED((NP,), jnp.float32),
        pltpu.VMEM_SHARED((NP,), jnp.float32),
        pltpu.VMEM_SHARED((NP,), jnp.float32),
        pltpu.VMEM_SHARED((NP,), jnp.float32),
        pltpu.VMEM_SHARED((NP,), jnp.float32),
        pltpu.VMEM((KC, LANES), jnp.int32),
        pltpu.VMEM((KC, LANES), jnp.int32),
        pltpu.VMEM((LANES,), jnp.float32),
        pltpu.VMEM((TN,), jnp.float32),
        pltpu.VMEM((TN,), jnp.float32),
        pltpu.VMEM((TN,), jnp.float32),
        pltpu.VMEM((TN,), jnp.float32),
        pltpu.VMEM((TN,), jnp.float32),
        pltpu.VMEM((LANES,), jnp.float32),
        pltpu.VMEM((LANES,), jnp.float32),
        pltpu.SemaphoreType.DMA,
        pltpu.SemaphoreType.DMA,
    ],
)
def _sc_aggregate(src_hbm, dst_hbm, xcols_hbm, z1_hbm,
                  agg_a_out, agg_b_out, dinv_out, dinv2_out,
                  deg_sh, ua_sh, ub_sh, aa_sh, ab_sh, st_src, st_dst, ones_v,
                  deg_l, dinv_l, dinv2_l, xa_l, xb_l, buf_a, buf_b,
                  sem_a, sem_b):
    c = lax.axis_index("c")
    s = lax.axis_index("s")
    base_n = s * TN
    base_r = s * TPT
    nsl = pl.ds(base_n, TN)

    pltpu.sync_copy(z1_hbm.at[nsl], deg_sh.at[nsl])
    pltpu.sync_copy(z1_hbm.at[nsl], aa_sh.at[nsl])
    pltpu.sync_copy(z1_hbm.at[nsl], ab_sh.at[nsl])
    for k in range(LANES // 16):
        ones_v[pl.ds(16 * k, 16)] = jnp.ones((16,), jnp.float32)
    plsc.subcore_barrier()

    def chunk_a(k, carry):
        r0 = base_r + k * KC
        pltpu.sync_copy(dst_hbm.at[c, pl.ds(r0, KC), :], st_dst)
        for j in range(KC):
            pltpu.sync_copy(ones_v, deg_sh.at[st_dst.at[j]], add=True)
        return carry

    lax.fori_loop(0, NCH, chunk_a, 0)
    plsc.subcore_barrier()

    pltpu.sync_copy(deg_sh.at[nsl], deg_l)
    pltpu.sync_copy(xcols_hbm.at[c, pl.ds(base_n, TN)], xa_l)
    pltpu.sync_copy(xcols_hbm.at[c, pl.ds(NP + base_n, TN)], xb_l)

    def newton(i, carry):
        sl = pl.ds(16 * i, 16)
        d = deg_l[sl] + 1.0
        bits = lax.bitcast_convert_type(d, jnp.int32)
        y = lax.bitcast_convert_type(jnp.int32(0x5F3759DF) - (bits >> 1),
                                     jnp.float32)
        hd = 0.5 * d
        y = y * (1.5 - hd * y * y)
        y = y * (1.5 - hd * y * y)
        y = y * (1.5 - hd * y * y)
        dinv_l[sl] = y
        dinv2_l[sl] = y * y
        xa_l[sl] = xa_l[sl] * y
        xb_l[sl] = xb_l[sl] * y
        return carry

    lax.fori_loop(0, TN // 16, newton, 0)

    pltpu.sync_copy(xa_l, ua_sh.at[nsl])
    pltpu.sync_copy(xb_l, ub_sh.at[nsl])
    pltpu.sync_copy(dinv_l, dinv_out.at[c, nsl])
    pltpu.sync_copy(dinv2_l, dinv2_out.at[c, nsl])
    plsc.subcore_barrier()

    def chunk_c(k, carry):
        r0 = base_r + k * KC
        pltpu.sync_copy(src_hbm.at[c, pl.ds(r0, KC), :], st_src)
        pltpu.sync_copy(dst_hbm.at[c, pl.ds(r0, KC), :], st_dst)
        for j in range(KC):
            ca = pltpu.async_copy(ua_sh.at[st_src.at[j]], buf_a, sem_a)
            cb = pltpu.async_copy(ub_sh.at[st_src.at[j]], buf_b, sem_b)
            ca.wait()
            cb.wait()
            pltpu.sync_copy(buf_a, aa_sh.at[st_dst.at[j]], add=True)
            pltpu.sync_copy(buf_b, ab_sh.at[st_dst.at[j]], add=True)
        return carry

    lax.fori_loop(0, NCH, chunk_c, 0)
    plsc.subcore_barrier()

    pltpu.sync_copy(aa_sh.at[nsl], agg_a_out.at[c, nsl])
    pltpu.sync_copy(ab_sh.at[nsl], agg_b_out.at[c, nsl])


@functools.partial(
    pl.kernel,
    out_type=jax.ShapeDtypeStruct((4, SP, H), jnp.float32),
    mesh=_sc_mesh,
    compiler_params=pltpu.CompilerParams(needs_layout_passes=False, use_tc_tiling_on_sc=False),
    scratch_types=[
        pltpu.VMEM((SPW,), jnp.int32),
        pltpu.VMEM((LANES, H), jnp.float32),
        pltpu.SemaphoreType.DMA,
    ],
)
def _sc_sample_gather(tab_hbm, sidx_hbm, g_hbm, idx_v, rows_v, sem):
    c = lax.axis_index("c")
    s = lax.axis_index("s")
    w = s * NC + c
    t = w // 8
    q = w % 8
    pltpu.sync_copy(sidx_hbm.at[t, pl.ds(q * SPW, SPW)], idx_v)

    def row(j, carry):
        pltpu.async_copy(
            tab_hbm.at[idx_v.at[pl.ds(j * LANES, LANES)]],
            rows_v, sem).wait()
        pltpu.sync_copy(
            rows_v, g_hbm.at[t, pl.ds(q * SPW + j * LANES, LANES), :])
        return carry

    lax.fori_loop(0, SPW // LANES, row, 0)


_RB = 1000
_GRID = N // _RB


def _cell_body(with_h, refs):
    if with_h:
        (agg_a, agg_b, dv, dv2, x, hp, wzrh, bzrh, lzw, lzb, lrw, lrb,
         lhw, lhb, out) = refs
    else:
        agg_a, agg_b, dv, dv2, x, wzrh, bzrh, lzw, lzb, lhw, lhb, out = refs
    d = dv[...]
    a = (jnp.concatenate([agg_a[...] * d, agg_b[...] * d], axis=1)
         + x[...] * dv2[...])
    cv = jnp.dot(a, wzrh[...], preferred_element_type=jnp.float32) + bzrh[...]
    cz = cv[:, 0:H]
    ch = cv[:, 2 * H:3 * H]
    lzw_v = lzw[...]
    lhw_v = lhw[...]
    if with_h:
        cr = cv[:, H:2 * H]
        lrw_v = lrw[...]
        h = hp[...]
        z = jax.nn.sigmoid(
            jnp.dot(cz, lzw_v[0:H], preferred_element_type=jnp.float32)
            + jnp.dot(h, lzw_v[H:2 * H], preferred_element_type=jnp.float32)
            + lzb[...])
        r = jax.nn.sigmoid(
            jnp.dot(cr, lrw_v[0:H], preferred_element_type=jnp.float32)
            + jnp.dot(h, lrw_v[H:2 * H], preferred_element_type=jnp.float32)
            + lrb[...])
        ht = jnp.tanh(
            jnp.dot(ch, lhw_v[0:H], preferred_element_type=jnp.float32)
            + jnp.dot(h * r, lhw_v[H:2 * H],
                      preferred_element_type=jnp.float32)
            + lhb[...])
        out[...] = z * h + (1.0 - z) * ht
    else:
        z = jax.nn.sigmoid(
            jnp.dot(cz, lzw_v[0:H], preferred_element_type=jnp.float32)
            + lzb[...])
        ht = jnp.tanh(
            jnp.dot(ch, lhw_v[0:H], preferred_element_type=jnp.float32)
            + lhb[...])
        out[...] = (1.0 - z) * ht


def _row_spec(width):
    return pl.BlockSpec((_RB, width), lambda i: (i, 0))


def _full_spec(shape):
    return pl.BlockSpec(shape, lambda i: (0, 0))


def _tgcn_cell0(agg_a, agg_b, dv, dv2, x, wzrh, bzrh, lzw, lzb, lhw, lhb):
    return pl.pallas_call(
        lambda *refs: _cell_body(False, refs),
        grid=(_GRID,),
        in_specs=[
            _row_spec(1), _row_spec(1), _row_spec(1), _row_spec(1),
            _row_spec(2),
            _full_spec((2, 3 * H)), _full_spec((1, 3 * H)),
            _full_spec((2 * H, H)), _full_spec((1, H)),
            _full_spec((2 * H, H)), _full_spec((1, H)),
        ],
        out_specs=_row_spec(H),
        out_shape=jax.ShapeDtypeStruct((N, H), jnp.float32),
    )(agg_a, agg_b, dv, dv2, x, wzrh, bzrh, lzw, lzb, lhw, lhb)


def _tgcn_cell1(agg_a, agg_b, dv, dv2, x, hp, wzrh, bzrh, lzw, lzb, lrw, lrb,
                lhw, lhb):
    return pl.pallas_call(
        lambda *refs: _cell_body(True, refs),
        grid=(_GRID,),
        in_specs=[
            _row_spec(1), _row_spec(1), _row_spec(1), _row_spec(1),
            _row_spec(2), _row_spec(H),
            _full_spec((2, 3 * H)), _full_spec((1, 3 * H)),
            _full_spec((2 * H, H)), _full_spec((1, H)),
            _full_spec((2 * H, H)), _full_spec((1, H)),
            _full_spec((2 * H, H)), _full_spec((1, H)),
        ],
        out_specs=_row_spec(H),
        out_shape=jax.ShapeDtypeStruct((N, H), jnp.float32),
    )(agg_a, agg_b, dv, dv2, x, hp, wzrh, bzrh, lzw, lzb, lrw, lrb, lhw, lhb)


_SB = 1000


def _cls_body(g0s, g0d, g1s, g1d, w1, b1, w2, b2, p0, p1):
    w1v = w1[...]
    b1v = b1[...]
    w2v = w2[...]
    b2v = b2[...]
    e0 = g0s[0] * g0d[0]
    h0 = jax.nn.relu(jnp.dot(e0, w1v, preferred_element_type=jnp.float32)
                     + b1v)
    p0[...] = jnp.dot(h0, w2v, preferred_element_type=jnp.float32) + b2v
    e1 = g1s[0] * g1d[0]
    h1 = jax.nn.relu(jnp.dot(e1, w1v, preferred_element_type=jnp.float32)
                     + b1v)
    p1[...] = jnp.dot(h1, w2v, preferred_element_type=jnp.float32) + b2v


def _classifier(g_all, w1, b1, w2, b2):
    gspec = lambda t: pl.BlockSpec((1, _SB, H), lambda i, t=t: (t, i, 0))
    return pl.pallas_call(
        _cls_body,
        grid=(S // _SB,),
        in_specs=[
            gspec(0), gspec(1), gspec(2), gspec(3),
            _full_spec((H, HID)), _full_spec((1, HID)),
            _full_spec((HID, 1)), _full_spec((1, 1)),
        ],
        out_specs=[
            pl.BlockSpec((_SB, 1), lambda i: (i, 0)),
            pl.BlockSpec((_SB, 1), lambda i: (i, 0)),
        ],
        out_shape=[
            jax.ShapeDtypeStruct((S, 1), jnp.float32),
            jax.ShapeDtypeStruct((S, 1), jnp.float32),
        ],
    )(g_all, g_all, g_all, g_all, w1, b1, w2, b2)


def kernel(x0, edge_index0, samples0, x1, edge_index1, samples1,
           W_z, b_z, W_r, b_r, W_h, b_h,
           lz_W, lz_b, lr_W, lr_b, lh_W, lh_b,
           fc1_W, fc1_b, fc2_W, fc2_b):
    f32 = jnp.float32

    ei = jnp.stack([edge_index0, edge_index1])
    pad = jnp.full((NC, 2, EP - E), NP - 1, jnp.int32)
    ei = jnp.concatenate([ei, pad], axis=2).reshape(NC, 2, ER, LANES)
    src_all = ei[:, 0]
    dst_all = ei[:, 1]

    xs = jnp.stack([x0, x1])
    xs = jnp.concatenate([xs, jnp.zeros((NC, NP - N, 2), f32)], axis=1)
    xcols = xs.transpose(0, 2, 1).reshape(NC, 2 * NP)

    z1 = jnp.zeros((NP,), f32)

    agg_a, agg_b, dv_all, dv2_all = _sc_aggregate(src_all, dst_all, xcols, z1)

    wzrh = jnp.concatenate([W_z, W_r, W_h], axis=1)
    bzrh = jnp.concatenate([b_z, b_r, b_h]).reshape(1, 3 * H)
    lzb = lz_b.reshape(1, H)
    lrb = lr_b.reshape(1, H)
    lhb = lh_b.reshape(1, H)

    out0 = _tgcn_cell0(agg_a[0].reshape(NP, 1), agg_b[0].reshape(NP, 1),
                       dv_all[0].reshape(NP, 1), dv2_all[0].reshape(NP, 1),
                       x0, wzrh, bzrh, lz_W, lzb, lh_W, lhb)
    out1 = _tgcn_cell1(agg_a[1].reshape(NP, 1), agg_b[1].reshape(NP, 1),
                       dv_all[1].reshape(NP, 1), dv2_all[1].reshape(NP, 1),
                       x1, out0, wzrh, bzrh, lz_W, lzb, lr_W, lrb, lh_W, lhb)

    sidx = jnp.concatenate([samples0.T, samples1.T + N])
    sidx = jnp.concatenate(
        [sidx, jnp.zeros((4, SP - S), jnp.int32)], axis=1)
    tab = jnp.concatenate([out0, out1])
    g_all = _sc_sample_gather(tab, sidx)

    pred0, pred1 = _classifier(
        g_all, fc1_W, fc1_b.reshape(1, HID), fc2_W, fc2_b.reshape(1, 1))

    return (pred0, pred1, out0, out1)


# speedup vs baseline: 98.7832x; 1.5066x over previous
import functools

import jax
import jax.numpy as jnp
from jax import lax
from jax.experimental import pallas as pl
from jax.experimental.pallas import tpu as pltpu
from jax.experimental.pallas import tpu_sc as plsc

N = 100000
E = 1600000
S = 10000
H = 32
HID = 128

NS = 16
NC = 2
TN = 6272
NP = NS * TN
EP = 1605632
ET = EP // NS
W = 1024
NCH = ET // W
SP = 10240
SPW = SP * 2 // 32

_SC_PARAMS = pltpu.CompilerParams(needs_layout_passes=False,
                                  use_tc_tiling_on_sc=False)


_sc_mesh = plsc.VectorSubcoreMesh(core_axis_name="c", subcore_axis_name="s")


@functools.partial(
    pl.kernel,
    out_type=jax.ShapeDtypeStruct((NC, 2 * NP), jnp.float32),
    mesh=_sc_mesh,
    compiler_params=_SC_PARAMS,
    scratch_types=[
        pltpu.VMEM_SHARED((NP,), jnp.float32),
        pltpu.VMEM_SHARED((NP,), jnp.float32),
        pltpu.VMEM_SHARED((NP,), jnp.float32),
        pltpu.VMEM_SHARED((NP,), jnp.float32),
        pltpu.VMEM_SHARED((NP,), jnp.float32),
        pltpu.VMEM((W,), jnp.int32),
        pltpu.VMEM((W,), jnp.int32),
        pltpu.VMEM((W,), jnp.float32),
        pltpu.VMEM((TN,), jnp.float32),
        pltpu.VMEM((TN,), jnp.float32),
        pltpu.VMEM((TN,), jnp.float32),
        pltpu.VMEM((TN,), jnp.float32),
        pltpu.VMEM((TN,), jnp.float32),
        pltpu.VMEM((2 * TN,), jnp.float32),
        pltpu.VMEM((W,), jnp.float32),
        pltpu.VMEM((W,), jnp.float32),
        pltpu.SemaphoreType.DMA,
        pltpu.SemaphoreType.DMA,
    ],
)
def _sc_aggregate(ei_hbm, xcols_hbm, z1_hbm, a_out,
                  deg_sh, ua_sh, ub_sh, aa_sh, ab_sh, st_src, st_dst, ones_v,
                  ta, tb, dinv_l, xa_l, xb_l, a2_l, buf_a, buf_b,
                  sem_a, sem_b):
    c = lax.axis_index("c")
    s = lax.axis_index("s")
    base_n = s * TN
    base_e = s * ET
    nsl = pl.ds(base_n, TN)

    pltpu.sync_copy(z1_hbm.at[nsl], deg_sh.at[nsl])
    pltpu.sync_copy(z1_hbm.at[nsl], aa_sh.at[nsl])
    pltpu.sync_copy(z1_hbm.at[nsl], ab_sh.at[nsl])

    def fill_ones(k, carry):
        ones_v[pl.ds(16 * k, 16)] = jnp.ones((16,), jnp.float32)
        return carry

    lax.fori_loop(0, W // 16, fill_ones, 0)
    plsc.subcore_barrier()

    def chunk_a(k, carry):
        off = base_e + k * W
        pltpu.sync_copy(ei_hbm.at[c, 1, pl.ds(off, W)], st_dst)
        pltpu.sync_copy(ones_v, deg_sh.at[st_dst], add=True)
        return carry

    lax.fori_loop(0, NCH, chunk_a, 0)
    plsc.subcore_barrier()

    pltpu.sync_copy(deg_sh.at[nsl], ta)
    pltpu.sync_copy(xcols_hbm.at[c, pl.ds(base_n, TN)], xa_l)
    pltpu.sync_copy(xcols_hbm.at[c, pl.ds(NP + base_n, TN)], xb_l)

    def newton(i, carry):
        sl = pl.ds(16 * i, 16)
        d = ta[sl] + 1.0
        bits = lax.bitcast_convert_type(d, jnp.int32)
        y = lax.bitcast_convert_type(jnp.int32(0x5F3759DF) - (bits >> 1),
                                     jnp.float32)
        hd = 0.5 * d
        y = y * (1.5 - hd * y * y)
        y = y * (1.5 - hd * y * y)
        y = y * (1.5 - hd * y * y)
        dinv_l[sl] = y
        xa_l[sl] = xa_l[sl] * y
        xb_l[sl] = xb_l[sl] * y
        return carry

    lax.fori_loop(0, TN // 16, newton, 0)

    pltpu.sync_copy(xa_l, ua_sh.at[nsl])
    pltpu.sync_copy(xb_l, ub_sh.at[nsl])
    plsc.subcore_barrier()

    def chunk_c(k, carry):
        off = base_e + k * W
        pltpu.sync_copy(ei_hbm.at[c, 0, pl.ds(off, W)], st_src)
        pltpu.sync_copy(ei_hbm.at[c, 1, pl.ds(off, W)], st_dst)
        ca = pltpu.async_copy(ua_sh.at[st_src], buf_a, sem_a)
        cb = pltpu.async_copy(ub_sh.at[st_src], buf_b, sem_b)
        ca.wait()
        cb.wait()
        pltpu.sync_copy(buf_a, aa_sh.at[st_dst], add=True)
        pltpu.sync_copy(buf_b, ab_sh.at[st_dst], add=True)
        return carry

    lax.fori_loop(0, NCH, chunk_c, 0)
    plsc.subcore_barrier()

    pltpu.sync_copy(aa_sh.at[nsl], ta)
    pltpu.sync_copy(ab_sh.at[nsl], tb)

    def finish(i, carry):
        sl = pl.ds(16 * i, 16)
        ta[sl] = dinv_l[sl] * (ta[sl] + xa_l[sl])
        tb[sl] = dinv_l[sl] * (tb[sl] + xb_l[sl])
        return carry

    lax.fori_loop(0, TN // 16, finish, 0)

    io = lax.iota(jnp.int32, 16)

    def interleave(i, carry):
        r16 = 8 * i + (io >> 1)
        va = plsc.load_gather(ta, [r16])
        vb = plsc.load_gather(tb, [r16])
        a2_l[pl.ds(16 * i, 16)] = jnp.where((io & 1) == 1, vb, va)
        return carry

    lax.fori_loop(0, 2 * TN // 16, interleave, 0)
    pltpu.sync_copy(a2_l, a_out.at[c, pl.ds(2 * base_n, 2 * TN)])


@functools.partial(
    pl.kernel,
    out_type=jax.ShapeDtypeStruct((2, SP, H), jnp.float32),
    mesh=_sc_mesh,
    compiler_params=_SC_PARAMS,
    scratch_types=[
        pltpu.VMEM((SPW,), jnp.int32),
        pltpu.VMEM((SPW, H), jnp.float32),
        pltpu.SemaphoreType.DMA,
    ],
)
def _sc_sample_gather(tab_hbm, sidx_hbm, g_hbm, idx_v, rows_v, sem):
    c = lax.axis_index("c")
    s = lax.axis_index("s")
    w = s * NC + c
    t = w // 16
    q = w % 16
    pltpu.sync_copy(sidx_hbm.at[t, pl.ds(q * SPW, SPW)], idx_v)
    pltpu.async_copy(tab_hbm.at[idx_v], rows_v, sem).wait()
    pltpu.sync_copy(rows_v, g_hbm.at[t, pl.ds(q * SPW, SPW), :])


_RB = 2000
_GRID = N // _RB


def _cell0_body(a_ref, wz, bz, wh, bh, lzw, lzb, lhw, lhb, out):
    a = a_ref[...]
    mz = jnp.dot(wz[...], lzw[...][0:H], preferred_element_type=jnp.float32)
    mh = jnp.dot(wh[...], lhw[...][0:H], preferred_element_type=jnp.float32)
    biz = jnp.dot(bz[...], lzw[...][0:H],
                  preferred_element_type=jnp.float32) + lzb[...]
    bih = jnp.dot(bh[...], lhw[...][0:H],
                  preferred_element_type=jnp.float32) + lhb[...]
    z = jax.nn.sigmoid(jnp.dot(a, mz, preferred_element_type=jnp.float32)
                       + biz)
    ht = jnp.tanh(jnp.dot(a, mh, preferred_element_type=jnp.float32) + bih)
    out[...] = (1.0 - z) * ht


def _cell1_body(a_ref, hp_ref, wz, bz, wr, br, wh, bh,
                lzw, lzb, lrw, lrb, lhw, lhb, out):
    a = a_ref[...]
    hp = hp_ref[...]
    lzw_v = lzw[...]
    lrw_v = lrw[...]
    lhw_v = lhw[...]
    mz = jnp.dot(wz[...], lzw_v[0:H], preferred_element_type=jnp.float32)
    mr = jnp.dot(wr[...], lrw_v[0:H], preferred_element_type=jnp.float32)
    mh = jnp.dot(wh[...], lhw_v[0:H], preferred_element_type=jnp.float32)
    m_zr = jnp.concatenate(
        [jnp.concatenate([mz, mr], axis=1),
         jnp.concatenate([lzw_v[H:], lrw_v[H:]], axis=1)], axis=0)
    b_zr = jnp.concatenate(
        [jnp.dot(bz[...], lzw_v[0:H], preferred_element_type=jnp.float32)
         + lzb[...],
         jnp.dot(br[...], lrw_v[0:H], preferred_element_type=jnp.float32)
         + lrb[...]], axis=1)
    ah = jnp.concatenate([a, hp], axis=1)
    zr = jax.nn.sigmoid(jnp.dot(ah, m_zr,
                                preferred_element_type=jnp.float32) + b_zr)
    z = zr[:, 0:H]
    r = zr[:, H:2 * H]
    m_h = jnp.concatenate([mh, lhw_v[H:]], axis=0)
    b_h = (jnp.dot(bh[...], lhw_v[0:H], preferred_element_type=jnp.float32)
           + lhb[...])
    ahr = jnp.concatenate([a, hp * r], axis=1)
    ht = jnp.tanh(jnp.dot(ahr, m_h, preferred_element_type=jnp.float32) + b_h)
    out[...] = z * hp + (1.0 - z) * ht


def _row_spec(width):
    return pl.BlockSpec((_RB, width), lambda i: (i, 0))


def _full_spec(shape):
    return pl.BlockSpec(shape, lambda i: (0,) * len(shape))


def _tgcn_cell0(a, wz, bz, wh, bh, lzw, lzb, lhw, lhb):
    return pl.pallas_call(
        _cell0_body,
        grid=(_GRID,),
        in_specs=[
            _row_spec(2),
            _full_spec((2, H)), _full_spec((1, H)),
            _full_spec((2, H)), _full_spec((1, H)),
            _full_spec((2 * H, H)), _full_spec((1, H)),
            _full_spec((2 * H, H)), _full_spec((1, H)),
        ],
        out_specs=_row_spec(H),
        out_shape=jax.ShapeDtypeStruct((N, H), jnp.float32),
    )(a, wz, bz, wh, bh, lzw, lzb, lhw, lhb)


def _tgcn_cell1(a, hp, wz, bz, wr, br, wh, bh, lzw, lzb, lrw, lrb, lhw, lhb):
    return pl.pallas_call(
        _cell1_body,
        grid=(_GRID,),
        in_specs=[
            _row_spec(2), _row_spec(H),
            _full_spec((2, H)), _full_spec((1, H)),
            _full_spec((2, H)), _full_spec((1, H)),
            _full_spec((2, H)), _full_spec((1, H)),
            _full_spec((2 * H, H)), _full_spec((1, H)),
            _full_spec((2 * H, H)), _full_spec((1, H)),
            _full_spec((2 * H, H)), _full_spec((1, H)),
        ],
        out_specs=_row_spec(H),
        out_shape=jax.ShapeDtypeStruct((N, H), jnp.float32),
    )(a, hp, wz, bz, wr, br, wh, bh, lzw, lzb, lrw, lrb, lhw, lhb)


_SB = 1000


def _cls_body(g0s, g0d, g1s, g1d, w1, b1, w2, b2, p0, p1):
    w1v = w1[...]
    b1v = b1[...]
    w2v = w2[...]
    b2v = b2[...]
    e0 = g0s[0] * g0d[0]
    h0 = jax.nn.relu(jnp.dot(e0, w1v, preferred_element_type=jnp.float32)
                     + b1v)
    p0[...] = jnp.dot(h0, w2v, preferred_element_type=jnp.float32) + b2v
    e1 = g1s[0] * g1d[0]
    h1 = jax.nn.relu(jnp.dot(e1, w1v, preferred_element_type=jnp.float32)
                     + b1v)
    p1[...] = jnp.dot(h1, w2v, preferred_element_type=jnp.float32) + b2v


def _classifier(g0, g1, w1, b1, w2, b2):
    gspec = lambda t: pl.BlockSpec((1, _SB, H), lambda i, t=t: (t, i, 0))
    return pl.pallas_call(
        _cls_body,
        grid=(S // _SB,),
        in_specs=[
            gspec(0), gspec(1), gspec(0), gspec(1),
            _full_spec((H, HID)), _full_spec((1, HID)),
            _full_spec((HID, 1)), _full_spec((1, 1)),
        ],
        out_specs=[
            pl.BlockSpec((_SB, 1), lambda i: (i, 0)),
            pl.BlockSpec((_SB, 1), lambda i: (i, 0)),
        ],
        out_shape=[
            jax.ShapeDtypeStruct((S, 1), jnp.float32),
            jax.ShapeDtypeStruct((S, 1), jnp.float32),
        ],
    )(g0, g0, g1, g1, w1, b1, w2, b2)


def kernel(x0, edge_index0, samples0, x1, edge_index1, samples1,
           W_z, b_z, W_r, b_r, W_h, b_h,
           lz_W, lz_b, lr_W, lr_b, lh_W, lh_b,
           fc1_W, fc1_b, fc2_W, fc2_b):
    f32 = jnp.float32

    ei = jnp.stack([edge_index0, edge_index1])
    pad = jnp.full((NC, 2, EP - E), NP - 1, jnp.int32)
    ei = jnp.concatenate([ei, pad], axis=2)

    xs = jnp.stack([x0, x1])
    xs = jnp.concatenate([xs, jnp.zeros((NC, NP - N, 2), f32)], axis=1)
    xcols = xs.transpose(0, 2, 1).reshape(NC, 2 * NP)

    z1 = jnp.zeros((NP,), f32)

    a_all = _sc_aggregate(ei, xcols, z1)
    a0 = a_all[0].reshape(NP, 2)
    a1 = a_all[1].reshape(NP, 2)

    bz = b_z.reshape(1, H)
    br = b_r.reshape(1, H)
    bh = b_h.reshape(1, H)
    lzb = lz_b.reshape(1, H)
    lrb = lr_b.reshape(1, H)
    lhb = lh_b.reshape(1, H)

    out0 = _tgcn_cell0(a0, W_z, bz, W_h, bh, lz_W, lzb, lh_W, lhb)
    out1 = _tgcn_cell1(a1, out0, W_z, bz, W_r, br, W_h, bh,
                       lz_W, lzb, lr_W, lrb, lh_W, lhb)

    zp = jnp.zeros((2, SP - S), jnp.int32)
    s0 = jnp.concatenate([samples0.T, zp], axis=1)
    s1 = jnp.concatenate([samples1.T, zp], axis=1)
    g0 = _sc_sample_gather(out0, s0)
    g1 = _sc_sample_gather(out1, s1)

    pred0, pred1 = _classifier(
        g0, g1, fc1_W, fc1_b.reshape(1, HID), fc2_W, fc2_b.reshape(1, 1))

    return (pred0, pred1, out0, out1)
